# baseline (speedup 1.0000x reference)
import functools

import jax
import jax.numpy as jnp
from jax import lax
from jax.experimental import pallas as pl
from jax.experimental.pallas import tpu as pltpu
from jax.experimental.pallas import tpu_sc as plsc

_F = 26
_V = 100000
_D = 32
_B = 1024
_S = 50

_N = _F * _B * _S
_NW = 32
_PER_W = _N // _NW
_K = 13
_RB = _K * 128
_NBLK = _PER_W // _RB


def _gather_body(table_hbm, gidx_hbm, out_hbm, idx_v, rows_v, sem):
    wid = lax.axis_index("s") * 2 + lax.axis_index("c")

    def block(i, carry):
        pltpu.sync_copy(gidx_hbm.at[wid, i], idx_v)
        pltpu.async_copy(table_hbm.at[idx_v], rows_v, sem).wait()
        pltpu.sync_copy(rows_v, out_hbm.at[wid, i])
        return carry

    lax.fori_loop(0, _NBLK, block, 0)


def kernel(ids, tables):
    offs = (jnp.arange(_F, dtype=jnp.int32) * _V)[:, None, None]
    gidx = (ids.astype(jnp.int32) + offs).transpose(1, 2, 0)
    gidx = gidx.reshape(_NW, _NBLK, _RB)
    table = tables.reshape(_F * _V, _D)

    mesh = plsc.VectorSubcoreMesh(core_axis_name="c", subcore_axis_name="s")
    out = pl.kernel(
        _gather_body,
        out_type=jax.ShapeDtypeStruct((_NW, _NBLK, _RB, _D), jnp.float32),
        mesh=mesh,
        scratch_types=[
            pltpu.VMEM((_RB,), jnp.int32),
            pltpu.VMEM((_RB, _D), jnp.float32),
            pltpu.SemaphoreType.DMA,
        ],
        compiler_params=pltpu.CompilerParams(use_tc_tiling_on_sc=False),
    )(table, gidx)

    out = out.reshape(_B, _S, _F * _D)
    seq_len = jnp.full((_B,), _S, dtype=ids.dtype)
    return out, seq_len


# speedup vs baseline: 1.0162x; 1.0162x over previous
import jax
import jax.numpy as jnp
from jax import lax
from jax.experimental import pallas as pl
from jax.experimental.pallas import tpu as pltpu
from jax.experimental.pallas import tpu_sc as plsc

_F = 26
_V = 100000
_D = 32
_B = 1024
_S = 50

_N = _F * _B * _S
_NW = 32
_PER_W = _N // _NW
_RB = 1600
_NBLK = _PER_W // _RB


def _gather_body(table_hbm, gidx_hbm, out_hbm,
                 idx0, idx1, rows0, rows1, gsem0, gsem1, ssem0, ssem1):
    wid = lax.axis_index("s") * 2 + lax.axis_index("c")

    def load_idx(i, idxv):
        pltpu.sync_copy(gidx_hbm.at[wid, i], idxv)

    def fire_gather(idxv, rowsv, sem):
        pltpu.async_copy(table_hbm.at[idxv], rowsv, sem)

    def fire_store(i, rowsv, sem):
        pltpu.async_copy(rowsv, out_hbm.at[wid, i], sem)

    def wait_gather(idxv, rowsv, sem):
        pltpu.make_async_copy(table_hbm.at[idxv], rowsv, sem).wait()

    def wait_store(i, rowsv, sem):
        pltpu.make_async_copy(rowsv, out_hbm.at[wid, i], sem).wait()

    load_idx(0, idx0)
    fire_gather(idx0, rows0, gsem0)
    load_idx(1, idx1)
    fire_gather(idx1, rows1, gsem1)
    wait_gather(idx0, rows0, gsem0)
    fire_store(0, rows0, ssem0)

    def body(m, carry):
        a = 2 * m + 2
        b = 2 * m + 3
        load_idx(a, idx0)
        wait_store(a - 2, rows0, ssem0)
        fire_gather(idx0, rows0, gsem0)
        wait_gather(idx1, rows1, gsem1)
        fire_store(a - 1, rows1, ssem1)
        load_idx(b, idx1)
        wait_store(a - 1, rows1, ssem1)
        fire_gather(idx1, rows1, gsem1)
        wait_gather(idx0, rows0, gsem0)
        fire_store(a, rows0, ssem0)
        return carry

    lax.fori_loop(0, (_NBLK - 2) // 2, body, 0)

    wait_gather(idx1, rows1, gsem1)
    fire_store(_NBLK - 1, rows1, ssem1)
    wait_store(_NBLK - 2, rows0, ssem0)
    wait_store(_NBLK - 1, rows1, ssem1)


def kernel(ids, tables):
    offs = (jnp.arange(_F, dtype=jnp.int32) * _V)[:, None, None]
    gidx = (ids.astype(jnp.int32) + offs).transpose(1, 2, 0)
    gidx = gidx.reshape(_NW, _NBLK, _RB)
    table = tables.reshape(_F * _V, _D)

    mesh = plsc.VectorSubcoreMesh(core_axis_name="c", subcore_axis_name="s")
    out = pl.kernel(
        _gather_body,
        out_type=jax.ShapeDtypeStruct((_NW, _NBLK, _RB, _D), jnp.float32),
        mesh=mesh,
        scratch_types=[
            pltpu.VMEM((_RB,), jnp.int32),
            pltpu.VMEM((_RB,), jnp.int32),
            pltpu.VMEM((_RB, _D), jnp.float32),
            pltpu.VMEM((_RB, _D), jnp.float32),
            pltpu.SemaphoreType.DMA,
            pltpu.SemaphoreType.DMA,
            pltpu.SemaphoreType.DMA,
            pltpu.SemaphoreType.DMA,
        ],
        compiler_params=pltpu.CompilerParams(use_tc_tiling_on_sc=False),
    )(table, gidx)

    out = out.reshape(_B, _S, _F * _D)
    seq_len = jnp.full((_B,), _S, dtype=ids.dtype)
    return out, seq_len


# speedup vs baseline: 1.0390x; 1.0224x over previous
import jax
import jax.numpy as jnp
from jax import lax
from jax.experimental import pallas as pl
from jax.experimental.pallas import tpu as pltpu
from jax.experimental.pallas import tpu_sc as plsc

_F = 26
_V = 100000
_D = 32
_B = 1024
_S = 50

_NW = 32
_TC = 5
_NCH = _S // _TC
_NVEC = _B // 16


def _gather_body(table_hbm, ids_hbm, out_hbm, tv, idx_v, ov, tsem, isem, osem):
    d = lax.axis_index("s") * 2 + lax.axis_index("c")

    def per_field(f, carry):
        pltpu.sync_copy(table_hbm.at[f, d], tv)
        fd = f * _D + d

        def per_chunk(c, carry2):
            t0 = c * _TC
            pltpu.sync_copy(ids_hbm.at[f, pl.ds(t0, _TC)], idx_v)

            def per_row(t, carry3):
                def per_vec(j, carry4):
                    lo = j * 16
                    vidx = idx_v[t, pl.ds(lo, 16)]
                    ov[t, pl.ds(lo, 16)] = plsc.load_gather(tv, [vidx])
                    return carry4

                lax.fori_loop(0, _NVEC, per_vec, 0)
                return carry3

            lax.fori_loop(0, _TC, per_row, 0)
            pltpu.sync_copy(ov, out_hbm.at[pl.ds(t0, _TC), fd])
            return carry2

        lax.fori_loop(0, _NCH, per_chunk, 0)
        return carry

    lax.fori_loop(0, _F, per_field, 0)


def kernel(ids, tables):
    ids_t = jnp.transpose(ids, (0, 2, 1))
    tables_t = jnp.transpose(tables, (0, 2, 1))

    mesh = plsc.VectorSubcoreMesh(core_axis_name="c", subcore_axis_name="s")
    out = pl.kernel(
        _gather_body,
        out_type=jax.ShapeDtypeStruct((_S, _F * _D, _B), jnp.float32),
        mesh=mesh,
        scratch_types=[
            pltpu.VMEM((_V,), jnp.float32),
            pltpu.VMEM((_TC, _B), jnp.int32),
            pltpu.VMEM((_TC, _B), jnp.float32),
            pltpu.SemaphoreType.DMA,
            pltpu.SemaphoreType.DMA,
            pltpu.SemaphoreType.DMA,
        ],
        compiler_params=pltpu.CompilerParams(
            use_tc_tiling_on_sc=False, needs_layout_passes=False
        ),
    )(tables_t, ids_t)

    out = jnp.transpose(out, (2, 0, 1))
    seq_len = jnp.full((_B,), _S, dtype=ids.dtype)
    return out, seq_len


# speedup vs baseline: 1.1510x; 1.1078x over previous
import jax
import jax.numpy as jnp
from jax import lax
from jax.experimental import pallas as pl
from jax.experimental.pallas import tpu as pltpu
from jax.experimental.pallas import tpu_sc as plsc

_F = 26
_V = 100000
_D = 32
_B = 1024
_S = 50

_NW = 32
_TC = 5
_NCH = _S // _TC
_NVEC = _B // 16


def _gather_body(table_hbm, ids_hbm, out_hbm, tv, idx_v, ov, tsem, isem, osem):
    d = lax.axis_index("s") * 2 + lax.axis_index("c")

    def per_field(f, carry):
        pltpu.sync_copy(table_hbm.at[f, d], tv)
        fd = f * _D + d

        def per_chunk(c, carry2):
            t0 = c * _TC
            pltpu.sync_copy(ids_hbm.at[f, pl.ds(t0, _TC)], idx_v)

            for t in range(_TC):
                for j in range(_NVEC):
                    lo = j * 16
                    vidx = idx_v[t, pl.ds(lo, 16)]
                    ov[t, pl.ds(lo, 16)] = plsc.load_gather(tv, [vidx])
            pltpu.sync_copy(ov, out_hbm.at[pl.ds(t0, _TC), fd])
            return carry2

        lax.fori_loop(0, _NCH, per_chunk, 0)
        return carry

    lax.fori_loop(0, _F, per_field, 0)


def kernel(ids, tables):
    ids_t = jnp.transpose(ids, (0, 2, 1))
    tables_t = jnp.transpose(tables, (0, 2, 1))

    mesh = plsc.VectorSubcoreMesh(core_axis_name="c", subcore_axis_name="s")
    out = pl.kernel(
        _gather_body,
        out_type=jax.ShapeDtypeStruct((_S, _F * _D, _B), jnp.float32),
        mesh=mesh,
        scratch_types=[
            pltpu.VMEM((_V,), jnp.float32),
            pltpu.VMEM((_TC, _B), jnp.int32),
            pltpu.VMEM((_TC, _B), jnp.float32),
            pltpu.SemaphoreType.DMA,
            pltpu.SemaphoreType.DMA,
            pltpu.SemaphoreType.DMA,
        ],
        compiler_params=pltpu.CompilerParams(
            use_tc_tiling_on_sc=False, needs_layout_passes=False
        ),
    )(tables_t, ids_t)

    out = jnp.transpose(out, (2, 0, 1))
    seq_len = jnp.full((_B,), _S, dtype=ids.dtype)
    return out, seq_len


# speedup vs baseline: 1.4250x; 1.2381x over previous
import jax
import jax.numpy as jnp
from jax import lax
from jax.experimental import pallas as pl
from jax.experimental.pallas import tpu as pltpu
from jax.experimental.pallas import tpu_sc as plsc

_F = 26
_V = 100000
_D = 32
_B = 1024
_S = 50

_NW = 32
_TC = 5
_NCH = _S // _TC
_NVEC = _B // 16


def _gather_body(table_hbm, ids_hbm, out_hbm,
                 tv, idx0, idx1, ov0, ov1, tsem, isem, osem):
    d = lax.axis_index("s") * 2 + lax.axis_index("c")
    idx_bufs = (idx0, idx1)
    ov_bufs = (ov0, ov1)

    def fire_tv(f):
        pltpu.async_copy(table_hbm.at[f, d], tv, tsem)

    def wait_tv(f):
        pltpu.make_async_copy(table_hbm.at[f, d], tv, tsem).wait()

    def fire_idx(f, c, buf):
        pltpu.async_copy(ids_hbm.at[f, pl.ds(c * _TC, _TC)], buf, isem)

    def wait_idx(f, c, buf):
        pltpu.make_async_copy(ids_hbm.at[f, pl.ds(c * _TC, _TC)], buf, isem).wait()

    def fire_store(fd, c, buf):
        pltpu.async_copy(buf, out_hbm.at[pl.ds(c * _TC, _TC), fd], osem)

    def wait_store(fd, c, buf):
        pltpu.make_async_copy(buf, out_hbm.at[pl.ds(c * _TC, _TC), fd], osem).wait()

    def gather_chunk(idx_v, ov):
        def per_row(t, carry):
            for j in range(_NVEC):
                lo = j * 16
                vidx = idx_v[t, pl.ds(lo, 16)]
                ov[t, pl.ds(lo, 16)] = plsc.load_gather(tv, [vidx])
            return carry

        lax.fori_loop(0, _TC, per_row, 0)

    fire_tv(0)
    fire_idx(0, 0, idx0)

    def per_field(f, carry):
        fd = f * _D + d
        wait_tv(f)
        for g in range(_NCH // 2):
            a, b = 2 * g, 2 * g + 1
            ia, ova = idx_bufs[0], ov_bufs[0]
            ib, ovb = idx_bufs[1], ov_bufs[1]

            wait_idx(f, a, ia)
            fire_idx(f, b, ib)
            if g == 0:
                @pl.when(f > 0)
                def _():
                    wait_store(fd - _D, _NCH - 2, ova)
            else:
                wait_store(fd, a - 2, ova)
            gather_chunk(ia, ova)
            fire_store(fd, a, ova)

            wait_idx(f, b, ib)
            if g < _NCH // 2 - 1:
                fire_idx(f, a + 2, ia)
            else:
                fire_idx(jnp.minimum(f + 1, _F - 1), 0, ia)
            if g == 0:
                @pl.when(f > 0)
                def _():
                    wait_store(fd - _D, _NCH - 1, ovb)
            else:
                wait_store(fd, b - 2, ovb)
            gather_chunk(ib, ovb)
            fire_store(fd, b, ovb)

        fire_tv(jnp.minimum(f + 1, _F - 1))
        return carry

    lax.fori_loop(0, _F, per_field, 0)

    wait_tv(_F - 1)
    wait_idx(_F - 1, 0, idx0)
    wait_store((_F - 1) * _D + d, _NCH - 2, ov0)
    wait_store((_F - 1) * _D + d, _NCH - 1, ov1)


def kernel(ids, tables):
    ids_t = jnp.transpose(ids, (0, 2, 1))
    tables_t = jnp.transpose(tables, (0, 2, 1))

    mesh = plsc.VectorSubcoreMesh(core_axis_name="c", subcore_axis_name="s")
    out = pl.kernel(
        _gather_body,
        out_type=jax.ShapeDtypeStruct((_S, _F * _D, _B), jnp.float32),
        mesh=mesh,
        scratch_types=[
            pltpu.VMEM((_V,), jnp.float32),
            pltpu.VMEM((_TC, _B), jnp.int32),
            pltpu.VMEM((_TC, _B), jnp.int32),
            pltpu.VMEM((_TC, _B), jnp.float32),
            pltpu.VMEM((_TC, _B), jnp.float32),
            pltpu.SemaphoreType.DMA,
            pltpu.SemaphoreType.DMA,
            pltpu.SemaphoreType.DMA,
        ],
        compiler_params=pltpu.CompilerParams(
            use_tc_tiling_on_sc=False, needs_layout_passes=False
        ),
    )(tables_t, ids_t)

    out = jnp.transpose(out, (2, 0, 1))
    seq_len = jnp.full((_B,), _S, dtype=ids.dtype)
    return out, seq_len


# speedup vs baseline: 1.4371x; 1.0085x over previous
import functools

import jax
import jax.numpy as jnp
from jax import lax
from jax.experimental import pallas as pl
from jax.experimental.pallas import tpu as pltpu
from jax.experimental.pallas import tpu_sc as plsc

_F = 26
_V = 100000
_D = 32
_B = 1024
_S = 50

_NW = 32
_TC = 5
_NCH = _S // _TC
_NVEC = _B // 16


def _gather_body(nf, table_hbm, ids_hbm, out_hbm,
                 tv, idx0, idx1, ov0, ov1, ids_sh, tsem, isem, osem, ssem):
    s = lax.axis_index("s")
    d = s * 2 + lax.axis_index("c")
    idx_bufs = (idx0, idx1)
    ov_bufs = (ov0, ov1)

    def fire_stage(f, b):
        @pl.when(s < _NCH)
        def _():
            pltpu.async_copy(ids_hbm.at[f, pl.ds(s * _TC, _TC)],
                             ids_sh.at[b, pl.ds(s * _TC, _TC)], ssem)

    def wait_stage(f, b):
        @pl.when(s < _NCH)
        def _():
            pltpu.make_async_copy(ids_hbm.at[f, pl.ds(s * _TC, _TC)],
                                  ids_sh.at[b, pl.ds(s * _TC, _TC)],
                                  ssem).wait()

    def fire_tv(f):
        pltpu.async_copy(table_hbm.at[f, d], tv, tsem)

    def wait_tv(f):
        pltpu.make_async_copy(table_hbm.at[f, d], tv, tsem).wait()

    def fire_idx(b, c, buf):
        pltpu.async_copy(ids_sh.at[b, pl.ds(c * _TC, _TC)], buf, isem)

    def wait_idx(b, c, buf):
        pltpu.make_async_copy(ids_sh.at[b, pl.ds(c * _TC, _TC)], buf,
                              isem).wait()

    def fire_store(fd, c, buf):
        pltpu.async_copy(buf, out_hbm.at[pl.ds(c * _TC, _TC), fd], osem)

    def wait_store(fd, c, buf):
        pltpu.make_async_copy(buf, out_hbm.at[pl.ds(c * _TC, _TC), fd], osem).wait()

    def gather_chunk(idx_v, ov):
        def per_row(t, carry):
            for j in range(_NVEC):
                lo = j * 16
                vidx = idx_v[t, pl.ds(lo, 16)]
                ov[t, pl.ds(lo, 16)] = plsc.load_gather(tv, [vidx])
            return carry

        lax.fori_loop(0, _TC, per_row, 0)

    fire_stage(0, 0)
    wait_stage(0, 0)
    plsc.subcore_barrier()
    fire_tv(0)

    def per_field(f, carry):
        fd = f * _D + d
        sb = lax.rem(f, 2)
        fire_stage(jnp.minimum(f + 1, nf - 1), 1 - sb)
        fire_idx(sb, 0, idx0)
        wait_tv(f)
        for g in range(_NCH // 2):
            a, b = 2 * g, 2 * g + 1
            ia, ova = idx_bufs[0], ov_bufs[0]
            ib, ovb = idx_bufs[1], ov_bufs[1]

            wait_idx(sb, a, ia)
            fire_idx(sb, b, ib)
            if g == 0:
                @pl.when(f > 0)
                def _():
                    wait_store(fd - _D, _NCH - 2, ova)
            else:
                wait_store(fd, a - 2, ova)
            gather_chunk(ia, ova)
            fire_store(fd, a, ova)

            wait_idx(sb, b, ib)
            if g < _NCH // 2 - 1:
                fire_idx(sb, a + 2, ia)
            if g == 0:
                @pl.when(f > 0)
                def _():
                    wait_store(fd - _D, _NCH - 1, ovb)
            else:
                wait_store(fd, b - 2, ovb)
            gather_chunk(ib, ovb)
            fire_store(fd, b, ovb)

        fire_tv(jnp.minimum(f + 1, nf - 1))
        wait_stage(jnp.minimum(f + 1, nf - 1), 1 - sb)
        plsc.subcore_barrier()
        return carry

    lax.fori_loop(0, nf, per_field, 0)

    wait_tv(nf - 1)
    wait_store((nf - 1) * _D + d, _NCH - 2, ov0)
    wait_store((nf - 1) * _D + d, _NCH - 1, ov1)


def kernel(ids, tables):
    ids_t = jnp.transpose(ids, (0, 2, 1))
    tables_t = jnp.transpose(tables, (0, 2, 1))

    mesh = plsc.VectorSubcoreMesh(core_axis_name="c", subcore_axis_name="s")
    out = pl.kernel(
        functools.partial(_gather_body, _F),
        out_type=jax.ShapeDtypeStruct((_S, _F * _D, _B), jnp.float32),
        mesh=mesh,
        scratch_types=[
            pltpu.VMEM((_V,), jnp.float32),
            pltpu.VMEM((_TC, _B), jnp.int32),
            pltpu.VMEM((_TC, _B), jnp.int32),
            pltpu.VMEM((_TC, _B), jnp.float32),
            pltpu.VMEM((_TC, _B), jnp.float32),
            pltpu.VMEM_SHARED((2, _S, _B), jnp.int32),
            pltpu.SemaphoreType.DMA,
            pltpu.SemaphoreType.DMA,
            pltpu.SemaphoreType.DMA,
            pltpu.SemaphoreType.DMA,
        ],
        compiler_params=pltpu.CompilerParams(
            use_tc_tiling_on_sc=False, needs_layout_passes=False
        ),
    )(tables_t, ids_t)

    out = jnp.transpose(out, (2, 0, 1))
    seq_len = jnp.full((_B,), _S, dtype=ids.dtype)
    return out, seq_len
